# baseline (speedup 1.0000x reference)
import functools
import math

import jax
import jax.numpy as jnp
from jax import lax
from jax.experimental import pallas as pl
from jax.experimental.pallas import tpu as pltpu
from jax.experimental.pallas import tpu_sc as plsc

D_MODEL = 64
VOCAB = 1000000
B, S = 4096, 200
TOTAL = B * S

NC, NS = 2, 16
NW = NC * NS
PER_W = TOTAL // NW

L = 128
K = 8
CH = K * L
IDX_ROWS = TOTAL // L
ROWS_PER_W = IDX_ROWS // NW
G = ROWS_PER_W // K

_mesh = plsc.VectorSubcoreMesh(core_axis_name="c", subcore_axis_name="s")


@functools.partial(
    pl.kernel,
    mesh=_mesh,
    out_type=jax.ShapeDtypeStruct((TOTAL, D_MODEL), jnp.float32),
    scratch_types=[
        pltpu.VMEM((K, L), jnp.int32),
        pltpu.VMEM((CH, D_MODEL), jnp.float32),
        pltpu.SemaphoreType.DMA,
    ],
    compiler_params=pltpu.CompilerParams(use_tc_tiling_on_sc=False),
)
def _emb_lookup(x_hbm, table_hbm, out_hbm, idx_v, rows_v, gsem):
    wid = lax.axis_index("s") * NC + lax.axis_index("c")
    row_base = wid * ROWS_PER_W

    def chunk(h, carry):
        row0 = row_base + h * K
        pltpu.sync_copy(x_hbm.at[pl.ds(row0, K)], idx_v)
        cps = [
            pltpu.async_copy(
                table_hbm.at[idx_v.at[j]],
                rows_v.at[pl.ds(j * L, L)],
                gsem,
            )
            for j in range(K)
        ]
        for cp in cps:
            cp.wait()

        def scale(r, c2):
            for q in range(D_MODEL // 16):
                sl = pl.ds(q * 16, 16)
                rows_v[r, sl] = rows_v[r, sl] * 8.0
            return c2

        lax.fori_loop(0, CH, scale, 0, unroll=2)

        pltpu.sync_copy(rows_v, out_hbm.at[pl.ds(row0 * L, CH)])
        return carry

    lax.fori_loop(0, G, chunk, 0)


def kernel(x, table):
    xf = x.reshape(IDX_ROWS, L).astype(jnp.int32)
    out = _emb_lookup(xf, table)
    return out.reshape(B, S, D_MODEL)


# speedup vs baseline: 1.0457x; 1.0457x over previous
import functools
import math

import jax
import jax.numpy as jnp
from jax import lax
from jax.experimental import pallas as pl
from jax.experimental.pallas import tpu as pltpu
from jax.experimental.pallas import tpu_sc as plsc

D_MODEL = 64
B, S = 4096, 200
TOTAL = B * S

NC, NS = 2, 16
NW = NC * NS

L = 128
K = 4
CH = K * L
IDX_ROWS = TOTAL // L
ROWS_PER_W = IDX_ROWS // NW
G = ROWS_PER_W // K
NBUF = 2

_mesh = plsc.VectorSubcoreMesh(core_axis_name="c", subcore_axis_name="s")


@functools.partial(
    pl.kernel,
    mesh=_mesh,
    out_type=jax.ShapeDtypeStruct((TOTAL, D_MODEL), jnp.float32),
    scratch_types=[
        pltpu.VMEM((NBUF, K, L), jnp.int32),
        pltpu.VMEM((NBUF, CH, D_MODEL), jnp.float32),
        [pltpu.SemaphoreType.DMA] * NBUF,
        [pltpu.SemaphoreType.DMA] * NBUF,
    ],
    compiler_params=pltpu.CompilerParams(use_tc_tiling_on_sc=False),
)
def _emb_lookup(x_hbm, table_hbm, out_hbm, idx_v, rows_v, gsems, wsems):
    wid = lax.axis_index("s") * NC + lax.axis_index("c")
    row_base = wid * ROWS_PER_W

    def start_gathers(h, b):
        pltpu.sync_copy(x_hbm.at[pl.ds(row_base + h * K, K)], idx_v.at[b])
        for j in range(K):
            pltpu.async_copy(
                table_hbm.at[idx_v.at[b].at[j]],
                rows_v.at[b].at[pl.ds(j * L, L)],
                gsems[b],
            )

    def drain_gathers(b):
        pltpu.make_async_copy(
            table_hbm.at[pl.ds(0, CH)], rows_v.at[b], gsems[b]
        ).wait()

    def drain_writeback(b):
        pltpu.make_async_copy(
            rows_v.at[b], out_hbm.at[pl.ds(0, CH)], wsems[b]
        ).wait()

    def scale(b):
        def body(r, c2):
            for q in range(D_MODEL // 16):
                sl = pl.ds(q * 16, 16)
                rows_v[b, r, sl] = rows_v[b, r, sl] * 8.0
            return c2

        lax.fori_loop(0, CH, body, 0, unroll=4)

    def process(h, b):
        @pl.when(h + 1 < G)
        def _prefetch():
            @pl.when(h >= 1)
            def _():
                drain_writeback(1 - b)

            start_gathers(h + 1, 1 - b)

        drain_gathers(b)
        scale(b)
        pltpu.async_copy(
            rows_v.at[b],
            out_hbm.at[pl.ds((row_base + h * K) * L, CH)],
            wsems[b],
        )

    start_gathers(0, 0)

    def outer(g, carry):
        for b in range(NBUF):
            process(g + b, b)
        return carry

    lax.fori_loop(0, G // NBUF, lambda i, c: outer(i * NBUF, c), 0)
    drain_writeback(NBUF - 2)
    drain_writeback(NBUF - 1)


def kernel(x, table):
    xf = x.reshape(IDX_ROWS, L).astype(jnp.int32)
    out = _emb_lookup(xf, table)
    return out.reshape(B, S, D_MODEL)


# speedup vs baseline: 1.3873x; 1.3266x over previous
import functools
import math

import jax
import jax.numpy as jnp
from jax import lax
from jax.experimental import pallas as pl
from jax.experimental.pallas import tpu as pltpu
from jax.experimental.pallas import tpu_sc as plsc

D_MODEL = 64
B, S = 4096, 200
TOTAL = B * S

NC, NS = 2, 16
NW = NC * NS

L = 128
K = 4
CH = K * L
IDX_ROWS = TOTAL // L
ROWS_PER_W = IDX_ROWS // NW
G = ROWS_PER_W // K
NBUF = 2

_mesh = plsc.VectorSubcoreMesh(core_axis_name="c", subcore_axis_name="s")


@functools.partial(
    pl.kernel,
    mesh=_mesh,
    out_type=jax.ShapeDtypeStruct((TOTAL, 128), jnp.float32),
    scratch_types=[
        pltpu.VMEM((NBUF, K, L), jnp.int32),
        pltpu.VMEM((NBUF, CH, D_MODEL), jnp.float32),
        [pltpu.SemaphoreType.DMA] * NBUF,
        [pltpu.SemaphoreType.DMA] * NBUF,
    ],
    compiler_params=pltpu.CompilerParams(use_tc_tiling_on_sc=False),
)
def _emb_lookup(x_hbm, table_hbm, out_hbm, idx_v, rows_v, gsems, wsems):
    wid = lax.axis_index("s") * NC + lax.axis_index("c")
    row_base = wid * ROWS_PER_W

    def start_gathers(h, b):
        pltpu.sync_copy(x_hbm.at[pl.ds(row_base + h * K, K)], idx_v.at[b])
        for j in range(K):
            pltpu.async_copy(
                table_hbm.at[idx_v.at[b].at[j]],
                rows_v.at[b].at[pl.ds(j * L, L)],
                gsems[b],
            )

    def drain_gathers(b):
        pltpu.make_async_copy(
            table_hbm.at[pl.ds(0, CH)], rows_v.at[b], gsems[b]
        ).wait()

    def drain_writeback(b):
        pltpu.make_async_copy(
            rows_v.at[b], out_hbm.at[pl.ds(0, CH), pl.ds(0, D_MODEL)], wsems[b]
        ).wait()

    def scale(b):
        def body(r, c2):
            for q in range(D_MODEL // 16):
                sl = pl.ds(q * 16, 16)
                rows_v[b, r, sl] = rows_v[b, r, sl] * 8.0
            return c2

        lax.fori_loop(0, CH, body, 0, unroll=4)

    def process(h, b):
        @pl.when(h + 1 < G)
        def _prefetch():
            @pl.when(h >= 1)
            def _():
                drain_writeback(1 - b)

            start_gathers(h + 1, 1 - b)

        drain_gathers(b)
        scale(b)
        pltpu.async_copy(
            rows_v.at[b],
            out_hbm.at[pl.ds((row_base + h * K) * L, CH), pl.ds(0, D_MODEL)],
            wsems[b],
        )

    start_gathers(0, 0)

    def outer(g, carry):
        for b in range(NBUF):
            process(g + b, b)
        return carry

    lax.fori_loop(0, G // NBUF, lambda i, c: outer(i * NBUF, c), 0)
    drain_writeback(NBUF - 2)
    drain_writeback(NBUF - 1)


def kernel(x, table):
    xf = x.reshape(IDX_ROWS, L).astype(jnp.int32)
    out = _emb_lookup(xf, table)
    return out.reshape(B, S, 128)[:, :, :D_MODEL]
